# Initial kernel scaffold; baseline (speedup 1.0000x reference)
import jax
import jax.numpy as jnp
from jax.experimental import pallas as pl


def kernel(x, supervision_edges, message_edges, message_edgewt, W_l1, W_r1, b1, W_l2, W_r2, b2, W_ewp, b_ewp, W_ep, b_ep):
    raise NotImplementedError("write your pallas kernel here")


# speedup vs baseline: 37.4464x; 37.4464x over previous
import functools

import jax
import jax.numpy as jnp
from jax import lax
from jax.experimental import pallas as pl
from jax.experimental.pallas import tpu as pltpu
from jax.experimental.pallas import tpu_sc as plsc

NC = 2
NS = 16
NW = NC * NS
LANES = 16
CH = 128
F = 128
FH = 64
AWH = 80
_HIGH = lax.Precision.HIGHEST


def _mesh():
    return plsc.VectorSubcoreMesh(core_axis_name="c", subcore_axis_name="s",
                                  num_cores=NC, num_subcores=NS)


def _make_agg(ncht, n):
    rpt = n // NS

    @functools.partial(
        pl.kernel,
        out_type=jax.ShapeDtypeStruct((NC, n, AWH), jnp.float32),
        mesh=_mesh(),
        compiler_params=pltpu.CompilerParams(use_tc_tiling_on_sc=False, needs_layout_passes=False),
        scratch_types=[
            pltpu.VMEM((ncht, CH), jnp.int32),
            pltpu.VMEM((ncht, CH), jnp.int32),
            pltpu.VMEM((ncht, CH), jnp.float32),
            pltpu.VMEM((CH, FH), jnp.float32),
            pltpu.VMEM((CH, AWH), jnp.float32),
            pltpu.VMEM_SHARED((n, AWH), jnp.float32),
        ],
    )
    def agg(xs_hbm, src_hbm, dst_hbm, wt_hbm, out_hbm,
            src_v, dst_v, wt_v, rows_v, aug_v, table):
        cid = lax.axis_index("c").astype(jnp.int32)
        sid = lax.axis_index("s").astype(jnp.int32)

        zeros16 = jnp.zeros((LANES,), jnp.float32)

        @pl.loop(0, CH)
        def _zero_aug(r):
            r = jnp.int32(r)
            for k in range(AWH // LANES):
                aug_v[r, pl.ds(k * LANES, LANES)] = zeros16

        base = sid * rpt
        nfull = rpt // CH
        rem = rpt - nfull * CH
        for i in range(nfull):
            pltpu.sync_copy(aug_v, table.at[pl.ds(base + i * CH, CH)])
        if rem:
            pltpu.sync_copy(aug_v.at[pl.ds(0, rem)],
                            table.at[pl.ds(base + nfull * CH, rem)])

        c0 = sid * ncht
        pltpu.sync_copy(src_hbm.at[pl.ds(c0, ncht)], src_v)
        pltpu.sync_copy(dst_hbm.at[pl.ds(c0, ncht)], dst_v)
        pltpu.sync_copy(wt_hbm.at[pl.ds(c0, ncht)], wt_v)

        off = cid * n

        @pl.loop(0, ncht)
        def _adj(c):
            c = jnp.int32(c)
            for g in range(CH // LANES):
                sl = pl.ds(g * LANES, LANES)
                src_v[c, sl] = src_v[c, sl] + off

        plsc.subcore_barrier()

        lanes = lax.iota(jnp.int32, LANES)
        col_w = lanes * 0 + FH

        @pl.loop(0, ncht)
        def _chunk(c):
            c = jnp.int32(c)

            pltpu.sync_copy(xs_hbm.at[src_v.at[c]], rows_v)

            @pl.loop(0, CH // LANES)
            def _grp(g):
                g = jnp.int32(g)
                wt16 = wt_v[c, pl.ds(g * LANES, LANES)]
                for j in range(LANES):
                    e = g * LANES + j
                    w = wt16[j]
                    for k in range(FH // LANES):
                        sl = pl.ds(k * LANES, LANES)
                        aug_v[e, sl] = rows_v[e, sl] * w
                plsc.store_scatter(aug_v, [lanes + g * LANES, col_w], wt16)

            pltpu.sync_copy(aug_v, table.at[dst_v.at[c]], add=True)

        plsc.subcore_barrier()

        pltpu.sync_copy(table.at[pl.ds(base, rpt)],
                        out_hbm.at[cid, pl.ds(base, rpt)])

    return agg


def _conv(p, x, wl, wr, b, block=1000):
    n = x.shape[0]

    def body(p_ref, x_ref, wl_ref, wr_ref, b_ref, hl_ref, hr_ref):
        wsum = jnp.maximum(p_ref[0, :, FH:FH + 1], 1e-12)
        num = jnp.concatenate([p_ref[0, :, :FH], p_ref[1, :, :FH]], axis=1)
        agg = num / wsum
        h = (jnp.dot(agg, wl_ref[...], precision=_HIGH,
                     preferred_element_type=jnp.float32)
             + jnp.dot(x_ref[...], wr_ref[...], precision=_HIGH,
                       preferred_element_type=jnp.float32)
             + b_ref[...])
        hl_ref[...] = h[:, :FH]
        hr_ref[...] = h[:, FH:]

    return pl.pallas_call(
        body,
        grid=(n // block,),
        in_specs=[
            pl.BlockSpec((NC, block, AWH), lambda i: (0, i, 0)),
            pl.BlockSpec((block, F), lambda i: (i, 0)),
            pl.BlockSpec((F, F), lambda i: (0, 0)),
            pl.BlockSpec((F, F), lambda i: (0, 0)),
            pl.BlockSpec((1, F), lambda i: (0, 0)),
        ],
        out_specs=[
            pl.BlockSpec((block, FH), lambda i: (i, 0)),
            pl.BlockSpec((block, FH), lambda i: (i, 0)),
        ],
        out_shape=[
            jax.ShapeDtypeStruct((n, FH), jnp.float32),
            jax.ShapeDtypeStruct((n, FH), jnp.float32),
        ],
    )(p, x, wl, wr, b)


def _conv2(p, xl, xr, wl, wr, b, wa1, wa2, bb1, bb2, block=1000):
    n = xl.shape[0]

    def body(p_ref, xl_ref, xr_ref, wl_ref, wr_ref, b_ref, wa1_ref, wa2_ref,
             bb1_ref, bb2_ref, h_ref, t1_ref, t2_ref):
        wsum = jnp.maximum(p_ref[0, :, FH:FH + 1], 1e-12)
        num = jnp.concatenate([p_ref[0, :, :FH], p_ref[1, :, :FH]], axis=1)
        agg = num / wsum
        x = jnp.concatenate([xl_ref[...], xr_ref[...]], axis=1)
        h = (jnp.dot(agg, wl_ref[...], precision=_HIGH,
                     preferred_element_type=jnp.float32)
             + jnp.dot(x, wr_ref[...], precision=_HIGH,
                       preferred_element_type=jnp.float32)
             + b_ref[...])
        h_ref[...] = h
        t1_ref[...] = jnp.dot(h, wa1_ref[...], precision=_HIGH,
                              preferred_element_type=jnp.float32) + bb1_ref[...]
        t2_ref[...] = jnp.dot(h, wa2_ref[...], precision=_HIGH,
                              preferred_element_type=jnp.float32) + bb2_ref[...]

    return pl.pallas_call(
        body,
        grid=(n // block,),
        in_specs=[
            pl.BlockSpec((NC, block, AWH), lambda i: (0, i, 0)),
            pl.BlockSpec((block, FH), lambda i: (i, 0)),
            pl.BlockSpec((block, FH), lambda i: (i, 0)),
            pl.BlockSpec((F, F), lambda i: (0, 0)),
            pl.BlockSpec((F, F), lambda i: (0, 0)),
            pl.BlockSpec((1, F), lambda i: (0, 0)),
            pl.BlockSpec((F, 1), lambda i: (0, 0)),
            pl.BlockSpec((F, 1), lambda i: (0, 0)),
            pl.BlockSpec((1, 1), lambda i: (0, 0)),
            pl.BlockSpec((1, 1), lambda i: (0, 0)),
        ],
        out_specs=[
            pl.BlockSpec((block, F), lambda i: (i, 0)),
            pl.BlockSpec((block, 1), lambda i: (i, 0)),
            pl.BlockSpec((block, 1), lambda i: (i, 0)),
        ],
        out_shape=[
            jax.ShapeDtypeStruct((n, F), jnp.float32),
            jax.ShapeDtypeStruct((n, 1), jnp.float32),
            jax.ShapeDtypeStruct((n, 1), jnp.float32),
        ],
    )(p, xl, xr, wl, wr, b, wa1, wa2, bb1, bb2)


def _make_score(nch, n):
    ep = NW * nch * CH
    ngrp = CH // LANES

    @functools.partial(
        pl.kernel,
        out_type=(jax.ShapeDtypeStruct((ep,), jnp.float32),
                  jax.ShapeDtypeStruct((ep,), jnp.float32)),
        mesh=_mesh(),
        compiler_params=pltpu.CompilerParams(use_tc_tiling_on_sc=False, needs_layout_passes=False),
        scratch_types=[
            pltpu.VMEM((nch, CH), jnp.int32),
            pltpu.VMEM((nch, CH), jnp.int32),
            pltpu.VMEM((n,), jnp.float32),
            pltpu.VMEM((n,), jnp.float32),
            pltpu.VMEM((F,), jnp.float32),
            pltpu.VMEM((F,), jnp.float32),
            pltpu.VMEM((CH, F), jnp.float32),
            pltpu.VMEM((CH, F), jnp.float32),
            pltpu.VMEM((2 * LANES * LANES,), jnp.float32),
            pltpu.VMEM((CH,), jnp.float32),
            pltpu.VMEM((CH,), jnp.float32),
        ],
    )
    def score(h_hbm, t1_hbm, t2_hbm, src_hbm, dst_hbm, wb1_hbm, wb2_hbm,
              o1_hbm, o2_hbm,
              src_v, dst_v, t1_v, t2_v, wb1_v, wb2_v, rows_s, rows_d,
              tsc, o1_v, o2_v):
        cid = lax.axis_index("c").astype(jnp.int32)
        sid = lax.axis_index("s").astype(jnp.int32)
        wid = sid * NC + cid
        c0 = wid * nch
        pltpu.sync_copy(src_hbm.at[pl.ds(c0, nch)], src_v)
        pltpu.sync_copy(dst_hbm.at[pl.ds(c0, nch)], dst_v)
        pltpu.sync_copy(t1_hbm, t1_v)
        pltpu.sync_copy(t2_hbm, t2_v)
        pltpu.sync_copy(wb1_hbm, wb1_v)
        pltpu.sync_copy(wb2_hbm, wb2_v)

        wb1 = [wb1_v[pl.ds(k * LANES, LANES)] for k in range(F // LANES)]
        wb2 = [wb2_v[pl.ds(k * LANES, LANES)] for k in range(F // LANES)]
        lcol = lax.iota(jnp.int32, LANES) * LANES

        @pl.loop(0, nch)
        def _chunk(c):
            c = jnp.int32(c)
            pltpu.sync_copy(h_hbm.at[src_v.at[c]], rows_s)
            pltpu.sync_copy(h_hbm.at[dst_v.at[c]], rows_d)

            @pl.loop(0, ngrp)
            def _grp(g):
                g = jnp.int32(g)
                for j in range(LANES):
                    e = g * LANES + j
                    acc1 = None
                    acc2 = None
                    for k in range(F // LANES):
                        sl = pl.ds(k * LANES, LANES)
                        prod = rows_s[e, sl] * rows_d[e, sl]
                        c1 = prod * wb1[k]
                        c2 = prod * wb2[k]
                        acc1 = c1 if acc1 is None else acc1 + c1
                        acc2 = c2 if acc2 is None else acc2 + c2
                    plsc.store_scatter(tsc, [lcol + j], acc1)
                    plsc.store_scatter(tsc, [lcol + j + LANES * LANES], acc2)

                r1 = tsc[pl.ds(0, LANES)]
                r2 = tsc[pl.ds(LANES * LANES, LANES)]
                for l in range(1, LANES):
                    r1 = r1 + tsc[pl.ds(l * LANES, LANES)]
                    r2 = r2 + tsc[pl.ds(LANES * LANES + l * LANES, LANES)]
                gsl = pl.ds(g * LANES, LANES)
                s16 = src_v[c, gsl]
                d16 = dst_v[c, gsl]
                t1e = plsc.load_gather(t1_v, [s16]) + plsc.load_gather(t1_v, [d16])
                t2e = plsc.load_gather(t2_v, [s16]) + plsc.load_gather(t2_v, [d16])
                o1_v[gsl] = jnp.maximum(r1 + t1e, 0.0)
                o2_v[gsl] = r2 + t2e

            ebase = (c0 + c) * CH
            pltpu.sync_copy(o1_v, o1_hbm.at[pl.ds(ebase, CH)])
            pltpu.sync_copy(o2_v, o2_hbm.at[pl.ds(ebase, CH)])

    return score


def _pad_reshape(a, nchunks, fill):
    ep = nchunks * CH
    pad = ep - a.shape[0]
    if pad:
        a = jnp.concatenate([a, jnp.full((pad,), fill, a.dtype)])
    return a.reshape(nchunks, CH)


def kernel(x, supervision_edges, message_edges, message_edgewt,
           W_l1, W_r1, b1, W_l2, W_r2, b2, W_ewp, b_ewp, W_ep, b_ep):
    with jax.enable_x64(False):
        o1, o2 = _impl(x, supervision_edges, message_edges, message_edgewt,
                       W_l1, W_r1, b1, W_l2, W_r2, b2, W_ewp, b_ewp, W_ep, b_ep)
    return (o1.astype(jnp.float64), o2.astype(jnp.float64))


def _impl(x, supervision_edges, message_edges, message_edgewt,
          W_l1, W_r1, b1, W_l2, W_r2, b2, W_ewp, b_ewp, W_ep, b_ep):
    n = x.shape[0]
    em = message_edges.shape[1]
    es = supervision_edges.shape[1]
    if message_edgewt is None:
        message_edgewt = jnp.ones((em,), jnp.float32)
    x = x.astype(jnp.float32)
    W_l1, W_r1, b1 = (w.astype(jnp.float32) for w in (W_l1, W_r1, b1))
    W_l2, W_r2, b2 = (w.astype(jnp.float32) for w in (W_l2, W_r2, b2))
    W_ewp, b_ewp = W_ewp.astype(jnp.float32), b_ewp.astype(jnp.float32)
    W_ep, b_ep = W_ep.astype(jnp.float32), b_ep.astype(jnp.float32)

    ncht_m = -(-em // (NS * CH))
    msrc = _pad_reshape(message_edges[0].astype(jnp.int32), NS * ncht_m, 0)
    mdst = _pad_reshape(message_edges[1].astype(jnp.int32), NS * ncht_m, 0)
    mwt = _pad_reshape(message_edgewt.astype(jnp.float32), NS * ncht_m, 0.0)
    nch_s = -(-es // (NW * CH))
    ssrc = _pad_reshape(supervision_edges[0].astype(jnp.int32), NW * nch_s, 0)
    sdst = _pad_reshape(supervision_edges[1].astype(jnp.int32), NW * nch_s, 0)

    xs = jnp.concatenate([x[:, :FH], x[:, FH:]], axis=0)
    p1 = _make_agg(ncht_m, n)(xs, msrc, mdst, mwt)
    hl1, hr1 = _conv(p1, x, W_l1, W_r1, b1.reshape(1, F))
    hs1 = jnp.concatenate([hl1, hr1], axis=0)
    p2 = _make_agg(ncht_m, n)(hs1, msrc, mdst, mwt)

    wa1 = W_ewp[:F]
    wb1 = W_ewp[F:, 0]
    wa2 = W_ep[:F]
    wb2 = W_ep[F:, 0]
    bb1 = (0.5 * b_ewp).reshape(1, 1)
    bb2 = (0.5 * b_ep).reshape(1, 1)
    h2, t1, t2 = _conv2(p2, hl1, hr1, W_l2, W_r2, b2.reshape(1, F),
                        wa1, wa2, bb1, bb2)

    o1, o2 = _make_score(nch_s, n)(h2, t1.reshape(-1), t2.reshape(-1),
                                   ssrc, sdst, wb1, wb2)
    return (o1[:es, None], o2[:es, None])


# speedup vs baseline: 46.5214x; 1.2423x over previous
import functools

import jax
import jax.numpy as jnp
from jax import lax
from jax.experimental import pallas as pl
from jax.experimental.pallas import tpu as pltpu
from jax.experimental.pallas import tpu_sc as plsc

NC = 2
NS = 16
NW = NC * NS
LANES = 16
CH = 128
F = 128
FH = 64
AWH = 80
_HIGH = lax.Precision.HIGHEST


def _mesh():
    return plsc.VectorSubcoreMesh(core_axis_name="c", subcore_axis_name="s",
                                  num_cores=NC, num_subcores=NS)


CHA = 64


def _make_agg(ncht, n):
    assert ncht % 2 == 0
    rpt = n // NS

    @functools.partial(
        pl.kernel,
        out_type=jax.ShapeDtypeStruct((NC, n, AWH), jnp.float32),
        mesh=_mesh(),
        compiler_params=pltpu.CompilerParams(use_tc_tiling_on_sc=False, needs_layout_passes=False),
        scratch_types=[
            pltpu.VMEM((ncht, CHA), jnp.int32),
            pltpu.VMEM((ncht, CHA), jnp.int32),
            pltpu.VMEM((ncht, CHA), jnp.float32),
            pltpu.VMEM((2, CHA, FH), jnp.float32),
            pltpu.VMEM((2, CHA, AWH), jnp.float32),
            pltpu.VMEM_SHARED((n, AWH), jnp.float32),
            pltpu.SemaphoreType.DMA,
            pltpu.SemaphoreType.DMA,
            pltpu.SemaphoreType.DMA,
            pltpu.SemaphoreType.DMA,
        ],
    )
    def agg(xs_hbm, src_hbm, dst_hbm, wt_hbm, out_hbm,
            src_v, dst_v, wt_v, rows_v, aug_v, table,
            gsem0, gsem1, ssem0, ssem1):
        cid = lax.axis_index("c").astype(jnp.int32)
        sid = lax.axis_index("s").astype(jnp.int32)
        gsems = (gsem0, gsem1)
        ssems = (ssem0, ssem1)

        zeros16 = jnp.zeros((LANES,), jnp.float32)

        @pl.loop(0, CHA)
        def _zero_aug(r):
            r = jnp.int32(r)
            for b in range(2):
                for k in range(AWH // LANES):
                    aug_v[b, r, pl.ds(k * LANES, LANES)] = zeros16

        base = sid * rpt
        nfull = rpt // CHA
        rem = rpt - nfull * CHA
        for i in range(nfull):
            pltpu.sync_copy(aug_v.at[0], table.at[pl.ds(base + i * CHA, CHA)])
        if rem:
            pltpu.sync_copy(aug_v.at[0, pl.ds(0, rem)],
                            table.at[pl.ds(base + nfull * CHA, rem)])

        c0 = sid * ncht
        pltpu.sync_copy(src_hbm.at[pl.ds(c0, ncht)], src_v)
        pltpu.sync_copy(dst_hbm.at[pl.ds(c0, ncht)], dst_v)
        pltpu.sync_copy(wt_hbm.at[pl.ds(c0, ncht)], wt_v)

        off = cid * n

        @pl.loop(0, ncht)
        def _adj(c):
            c = jnp.int32(c)
            for g in range(CHA // LANES):
                sl = pl.ds(g * LANES, LANES)
                src_v[c, sl] = src_v[c, sl] + off

        plsc.subcore_barrier()

        lanes = lax.iota(jnp.int32, LANES)
        col_w = lanes * 0 + FH

        pltpu.async_copy(xs_hbm.at[src_v.at[0]], rows_v.at[0], gsem0)
        pltpu.async_copy(xs_hbm.at[src_v.at[1]], rows_v.at[1], gsem1)

        @pl.loop(0, ncht, step=2)
        def _c(c):
            c = jnp.int32(c)
            for b in range(2):
                cc = c + b
                pltpu.make_async_copy(xs_hbm.at[src_v.at[cc]],
                                      rows_v.at[b], gsems[b]).wait()

                @pl.when(cc >= 2)
                def _drain():
                    pltpu.make_async_copy(aug_v.at[b], table.at[dst_v.at[cc]],
                                          ssems[b]).wait()

                @pl.loop(0, CHA // LANES)
                def _grp(g):
                    g = jnp.int32(g)
                    gsl = pl.ds(g * LANES, LANES)
                    wt16 = wt_v[cc, gsl]
                    for j in range(LANES):
                        e = g * LANES + j
                        w = wt16[j]
                        for k in range(FH // LANES):
                            sl = pl.ds(k * LANES, LANES)
                            aug_v[b, e, sl] = rows_v[b, e, sl] * w
                    plsc.store_scatter(aug_v.at[b],
                                       [lanes + g * LANES, col_w], wt16)

                pltpu.async_copy(aug_v.at[b], table.at[dst_v.at[cc]],
                                 ssems[b], add=True)

                @pl.when(cc + 2 < ncht)
                def _next():
                    pltpu.async_copy(xs_hbm.at[src_v.at[cc + 2]],
                                     rows_v.at[b], gsems[b])

        for b in range(2):
            pltpu.make_async_copy(aug_v.at[b], table.at[dst_v.at[b]],
                                  ssems[b]).wait()

        plsc.subcore_barrier()

        pltpu.sync_copy(table.at[pl.ds(base, rpt)],
                        out_hbm.at[cid, pl.ds(base, rpt)])

    return agg


def _conv(p, x, wl, wr, b, block=1000):
    n = x.shape[0]

    def body(p_ref, x_ref, wl_ref, wr_ref, b_ref, hl_ref, hr_ref):
        wsum = jnp.maximum(p_ref[0, :, FH:FH + 1], 1e-12)
        num = jnp.concatenate([p_ref[0, :, :FH], p_ref[1, :, :FH]], axis=1)
        agg = num / wsum
        h = (jnp.dot(agg, wl_ref[...], precision=_HIGH,
                     preferred_element_type=jnp.float32)
             + jnp.dot(x_ref[...], wr_ref[...], precision=_HIGH,
                       preferred_element_type=jnp.float32)
             + b_ref[...])
        hl_ref[...] = h[:, :FH]
        hr_ref[...] = h[:, FH:]

    return pl.pallas_call(
        body,
        grid=(n // block,),
        in_specs=[
            pl.BlockSpec((NC, block, AWH), lambda i: (0, i, 0)),
            pl.BlockSpec((block, F), lambda i: (i, 0)),
            pl.BlockSpec((F, F), lambda i: (0, 0)),
            pl.BlockSpec((F, F), lambda i: (0, 0)),
            pl.BlockSpec((1, F), lambda i: (0, 0)),
        ],
        out_specs=[
            pl.BlockSpec((block, FH), lambda i: (i, 0)),
            pl.BlockSpec((block, FH), lambda i: (i, 0)),
        ],
        out_shape=[
            jax.ShapeDtypeStruct((n, FH), jnp.float32),
            jax.ShapeDtypeStruct((n, FH), jnp.float32),
        ],
    )(p, x, wl, wr, b)


def _conv2(p, xl, xr, wl, wr, b, wa1, wa2, bb1, bb2, block=1000):
    n = xl.shape[0]

    def body(p_ref, xl_ref, xr_ref, wl_ref, wr_ref, b_ref,
             wa1_ref, wa2_ref, bb1_ref, bb2_ref, h_ref, t1_ref, t2_ref):
        wsum = jnp.maximum(p_ref[0, :, FH:FH + 1], 1e-12)
        num = jnp.concatenate([p_ref[0, :, :FH], p_ref[1, :, :FH]], axis=1)
        agg = num / wsum
        x = jnp.concatenate([xl_ref[...], xr_ref[...]], axis=1)
        h = (jnp.dot(agg, wl_ref[...], precision=_HIGH,
                     preferred_element_type=jnp.float32)
             + jnp.dot(x, wr_ref[...], precision=_HIGH,
                       preferred_element_type=jnp.float32)
             + b_ref[...])
        h_ref[...] = h
        t1_ref[...] = jnp.dot(h, wa1_ref[...], precision=_HIGH,
                              preferred_element_type=jnp.float32) + bb1_ref[...]
        t2_ref[...] = jnp.dot(h, wa2_ref[...], precision=_HIGH,
                              preferred_element_type=jnp.float32) + bb2_ref[...]

    return pl.pallas_call(
        body,
        grid=(n // block,),
        in_specs=[
            pl.BlockSpec((NC, block, AWH), lambda i: (0, i, 0)),
            pl.BlockSpec((block, FH), lambda i: (i, 0)),
            pl.BlockSpec((block, FH), lambda i: (i, 0)),
            pl.BlockSpec((F, F), lambda i: (0, 0)),
            pl.BlockSpec((F, F), lambda i: (0, 0)),
            pl.BlockSpec((1, F), lambda i: (0, 0)),
            pl.BlockSpec((F, 1), lambda i: (0, 0)),
            pl.BlockSpec((F, 1), lambda i: (0, 0)),
            pl.BlockSpec((1, 1), lambda i: (0, 0)),
            pl.BlockSpec((1, 1), lambda i: (0, 0)),
        ],
        out_specs=[
            pl.BlockSpec((block, F), lambda i: (i, 0)),
            pl.BlockSpec((block, 1), lambda i: (i, 0)),
            pl.BlockSpec((block, 1), lambda i: (i, 0)),
        ],
        out_shape=[
            jax.ShapeDtypeStruct((n, F), jnp.float32),
            jax.ShapeDtypeStruct((n, 1), jnp.float32),
            jax.ShapeDtypeStruct((n, 1), jnp.float32),
        ],
    )(p, xl, xr, wl, wr, b, wa1, wa2, bb1, bb2)


def _make_score(nch, n):
    assert nch % 2 == 0
    ep = NW * nch * CH
    ngrp = CH // LANES

    @functools.partial(
        pl.kernel,
        out_type=(jax.ShapeDtypeStruct((ep,), jnp.float32),
                  jax.ShapeDtypeStruct((ep,), jnp.float32)),
        mesh=_mesh(),
        compiler_params=pltpu.CompilerParams(use_tc_tiling_on_sc=False, needs_layout_passes=False),
        scratch_types=[
            pltpu.VMEM((nch, CH), jnp.int32),
            pltpu.VMEM((nch, CH), jnp.int32),
            pltpu.VMEM((n,), jnp.float32),
            pltpu.VMEM((n,), jnp.float32),
            pltpu.VMEM((F,), jnp.float32),
            pltpu.VMEM((F,), jnp.float32),
            pltpu.VMEM((2, CH, F), jnp.float32),
            pltpu.VMEM((2, CH, F), jnp.float32),
            pltpu.VMEM((2 * LANES * LANES,), jnp.float32),
            pltpu.VMEM((2, CH), jnp.float32),
            pltpu.VMEM((2, CH), jnp.float32),
            pltpu.SemaphoreType.DMA,
            pltpu.SemaphoreType.DMA,
            pltpu.SemaphoreType.DMA,
            pltpu.SemaphoreType.DMA,
            pltpu.SemaphoreType.DMA,
            pltpu.SemaphoreType.DMA,
        ],
    )
    def score(h_hbm, t1_hbm, t2_hbm, src_hbm, dst_hbm, wb1_hbm, wb2_hbm,
              o1_hbm, o2_hbm,
              src_v, dst_v, t1_v, t2_v, wb1_v, wb2_v, rows_s, rows_d,
              tsc, o1_v, o2_v, gs0, gs1, gd0, gd1, os0, os1):
        cid = lax.axis_index("c").astype(jnp.int32)
        sid = lax.axis_index("s").astype(jnp.int32)
        wid = sid * NC + cid
        c0 = wid * nch
        gss = (gs0, gs1)
        gds = (gd0, gd1)
        oss = (os0, os1)
        pltpu.sync_copy(src_hbm.at[pl.ds(c0, nch)], src_v)
        pltpu.sync_copy(dst_hbm.at[pl.ds(c0, nch)], dst_v)
        pltpu.sync_copy(t1_hbm, t1_v)
        pltpu.sync_copy(t2_hbm, t2_v)
        pltpu.sync_copy(wb1_hbm, wb1_v)
        pltpu.sync_copy(wb2_hbm, wb2_v)

        wb1 = [wb1_v[pl.ds(k * LANES, LANES)] for k in range(F // LANES)]
        wb2 = [wb2_v[pl.ds(k * LANES, LANES)] for k in range(F // LANES)]
        lcol = lax.iota(jnp.int32, LANES) * LANES

        for b in range(2):
            pltpu.async_copy(h_hbm.at[src_v.at[b]], rows_s.at[b], gss[b])
            pltpu.async_copy(h_hbm.at[dst_v.at[b]], rows_d.at[b], gds[b])

        @pl.loop(0, nch, step=2)
        def _c(c):
            c = jnp.int32(c)
            for b in range(2):
                cc = c + b
                ebase = (c0 + cc) * CH
                pltpu.make_async_copy(h_hbm.at[src_v.at[cc]],
                                      rows_s.at[b], gss[b]).wait()
                pltpu.make_async_copy(h_hbm.at[dst_v.at[cc]],
                                      rows_d.at[b], gds[b]).wait()

                @pl.when(cc >= 2)
                def _drain():
                    eprev = ebase - 2 * CH
                    pltpu.make_async_copy(o1_v.at[b],
                                          o1_hbm.at[pl.ds(eprev, CH)],
                                          oss[b]).wait()
                    pltpu.make_async_copy(o2_v.at[b],
                                          o2_hbm.at[pl.ds(eprev, CH)],
                                          oss[b]).wait()

                @pl.loop(0, ngrp)
                def _grp(g):
                    g = jnp.int32(g)
                    for j in range(LANES):
                        e = g * LANES + j
                        acc1 = None
                        acc2 = None
                        for k in range(F // LANES):
                            sl = pl.ds(k * LANES, LANES)
                            prod = rows_s[b, e, sl] * rows_d[b, e, sl]
                            c1 = prod * wb1[k]
                            c2 = prod * wb2[k]
                            acc1 = c1 if acc1 is None else acc1 + c1
                            acc2 = c2 if acc2 is None else acc2 + c2
                        plsc.store_scatter(tsc, [lcol + j], acc1)
                        plsc.store_scatter(tsc, [lcol + j + LANES * LANES], acc2)

                    r1 = tsc[pl.ds(0, LANES)]
                    r2 = tsc[pl.ds(LANES * LANES, LANES)]
                    for l in range(1, LANES):
                        r1 = r1 + tsc[pl.ds(l * LANES, LANES)]
                        r2 = r2 + tsc[pl.ds(LANES * LANES + l * LANES, LANES)]
                    gsl = pl.ds(g * LANES, LANES)
                    s16 = src_v[cc, gsl]
                    d16 = dst_v[cc, gsl]
                    t1e = plsc.load_gather(t1_v, [s16]) + plsc.load_gather(t1_v, [d16])
                    t2e = plsc.load_gather(t2_v, [s16]) + plsc.load_gather(t2_v, [d16])
                    o1_v[b, gsl] = jnp.maximum(r1 + t1e, 0.0)
                    o2_v[b, gsl] = r2 + t2e

                pltpu.async_copy(o1_v.at[b], o1_hbm.at[pl.ds(ebase, CH)], oss[b])
                pltpu.async_copy(o2_v.at[b], o2_hbm.at[pl.ds(ebase, CH)], oss[b])

                @pl.when(cc + 2 < nch)
                def _next():
                    pltpu.async_copy(h_hbm.at[src_v.at[cc + 2]],
                                     rows_s.at[b], gss[b])
                    pltpu.async_copy(h_hbm.at[dst_v.at[cc + 2]],
                                     rows_d.at[b], gds[b])

        for b in range(2):
            elast = (c0 + nch - 2 + b) * CH
            pltpu.make_async_copy(o1_v.at[b], o1_hbm.at[pl.ds(elast, CH)],
                                  oss[b]).wait()
            pltpu.make_async_copy(o2_v.at[b], o2_hbm.at[pl.ds(elast, CH)],
                                  oss[b]).wait()

    return score


def _pad_reshape(a, nchunks, fill, ch):
    ep = nchunks * ch
    pad = ep - a.shape[0]
    if pad:
        a = jnp.concatenate([a, jnp.full((pad,), fill, a.dtype)])
    return a.reshape(nchunks, ch)


def kernel(x, supervision_edges, message_edges, message_edgewt,
           W_l1, W_r1, b1, W_l2, W_r2, b2, W_ewp, b_ewp, W_ep, b_ep):
    with jax.enable_x64(False):
        o1, o2 = _impl(x, supervision_edges, message_edges, message_edgewt,
                       W_l1, W_r1, b1, W_l2, W_r2, b2, W_ewp, b_ewp, W_ep, b_ep)
    return (o1.astype(jnp.float64), o2.astype(jnp.float64))


def _impl(x, supervision_edges, message_edges, message_edgewt,
          W_l1, W_r1, b1, W_l2, W_r2, b2, W_ewp, b_ewp, W_ep, b_ep):
    n = x.shape[0]
    em = message_edges.shape[1]
    es = supervision_edges.shape[1]
    if message_edgewt is None:
        message_edgewt = jnp.ones((em,), jnp.float32)
    x = x.astype(jnp.float32)
    W_l1, W_r1, b1 = (w.astype(jnp.float32) for w in (W_l1, W_r1, b1))
    W_l2, W_r2, b2 = (w.astype(jnp.float32) for w in (W_l2, W_r2, b2))
    W_ewp, b_ewp = W_ewp.astype(jnp.float32), b_ewp.astype(jnp.float32)
    W_ep, b_ep = W_ep.astype(jnp.float32), b_ep.astype(jnp.float32)

    ncht_m = -(-em // (NS * CHA))
    ncht_m += ncht_m % 2
    msrc = _pad_reshape(message_edges[0].astype(jnp.int32), NS * ncht_m, 0, CHA)
    mdst = _pad_reshape(message_edges[1].astype(jnp.int32), NS * ncht_m, 0, CHA)
    mwt = _pad_reshape(message_edgewt.astype(jnp.float32), NS * ncht_m, 0.0, CHA)
    nch_s = -(-es // (NW * CH))
    nch_s += nch_s % 2
    ssrc = _pad_reshape(supervision_edges[0].astype(jnp.int32), NW * nch_s, 0, CH)
    sdst = _pad_reshape(supervision_edges[1].astype(jnp.int32), NW * nch_s, 0, CH)

    xs = jnp.concatenate([x[:, :FH], x[:, FH:]], axis=0)
    p1 = _make_agg(ncht_m, n)(xs, msrc, mdst, mwt)
    hl1, hr1 = _conv(p1, x, W_l1, W_r1, b1.reshape(1, F))
    hs1 = jnp.concatenate([hl1, hr1], axis=0)
    p2 = _make_agg(ncht_m, n)(hs1, msrc, mdst, mwt)

    wa1 = W_ewp[:F]
    wb1 = W_ewp[F:, 0]
    wa2 = W_ep[:F]
    wb2 = W_ep[F:, 0]
    bb1 = (0.5 * b_ewp).reshape(1, 1)
    bb2 = (0.5 * b_ep).reshape(1, 1)
    h2, t1, t2 = _conv2(p2, hl1, hr1, W_l2, W_r2, b2.reshape(1, F),
                        wa1, wa2, bb1, bb2)

    o1, o2 = _make_score(nch_s, n)(h2, t1.reshape(-1), t2.reshape(-1),
                                   ssrc, sdst, wb1, wb2)
    return (o1[:es, None], o2[:es, None])
